# baseline (speedup 1.0000x reference)
import jax
import jax.numpy as jnp
from jax import lax
from jax.experimental import pallas as pl
from jax.experimental.pallas import tpu as pltpu
from jax.experimental.pallas import tpu_sc as plsc

N = 10000
E = 320000
F_IN = 64
F_OUT = 128

NC = 2
NS = 16
N_PAD = 10240

EPT = E // NS
DEG_K = 2000
DEG_CHUNKS = EPT // DEG_K
EK = 200
ECHUNKS = EPT // EK
SK = 200

RT = 600
WB_TILES = 10
WB_ROWS = N // WB_TILES

_mesh = plsc.VectorSubcoreMesh(
    core_axis_name="c", subcore_axis_name="s", num_cores=NC, num_subcores=NS)

_sc_params = pltpu.CompilerParams(use_tc_tiling_on_sc=False,
                                  needs_layout_passes=False)


def _newton_rsqrt16(v):
    x = jnp.maximum(v, 1.0)
    i = plsc.bitcast(x, jnp.int32)
    i = jnp.int32(0x5F3759DF) - lax.shift_right_logical(i, 1)
    y = plsc.bitcast(i, jnp.float32)
    for _ in range(3):
        y = y * (1.5 - 0.5 * x * y * y)
    return y


def _fused_body(feats0, feats1, edges0, edges1, zeros1d, zeros2d,
                agg0_out, agg1_out, indeg0_out, indeg1_out,
                es_v, ed_v, rows_v, sidx, didx, ones_v, nrm_v,
                h_sh, agg_sh, sdeg_sh, ddeg_sh,
                si_s, si_d, sg, ss, dsi_s, dsi_d, dsa_s, dsa_d):
    c = lax.axis_index("c")
    s = lax.axis_index("s")
    base = s * EPT

    def percore(fn):
        @pl.when(c == 0)
        def _():
            fn(feats0, edges0, agg0_out, indeg0_out)

        @pl.when(c == 1)
        def _():
            fn(feats1, edges1, agg1_out, indeg1_out)

    def dg_src_d(egs, ch, b, sem):
        return pltpu.make_async_copy(
            egs.at[0, pl.ds(base + ch * DEG_K, DEG_K)], sidx.at[b], sem)

    def dg_dst_d(egs, ch, b, sem):
        return pltpu.make_async_copy(
            egs.at[1, pl.ds(base + ch * DEG_K, DEG_K)], didx.at[b], sem)

    def _deg_prologue(fts, egs, agg_o, ind_o):
        for b in range(2):
            dg_src_d(egs, b, b, dsi_s.at[b]).start()
            dg_dst_d(egs, b, b, dsi_d.at[b]).start()

    percore(_deg_prologue)

    zn = N_PAD // NS
    z0 = s * zn
    pltpu.sync_copy(zeros1d.at[pl.ds(z0, zn)], sdeg_sh.at[pl.ds(z0, zn)])
    pltpu.sync_copy(zeros1d.at[pl.ds(z0, zn)], ddeg_sh.at[pl.ds(z0, zn)])

    @pl.when(s < WB_TILES)
    def _():
        r0 = s * WB_ROWS
        pltpu.sync_copy(zeros2d.at[pl.ds(r0, WB_ROWS)],
                        agg_sh.at[pl.ds(r0, WB_ROWS)])

    @pl.loop(0, DEG_K, step=16)
    def _(i):
        ones_v[pl.ds(i, 16)] = jnp.full((16,), 1.0, jnp.float32)

    plsc.subcore_barrier()

    def deg_chunk(ch, b, prefetch, guard):
        dg_src_d(edges0, ch, b, dsi_s.at[b]).wait()
        dg_dst_d(edges0, ch, b, dsi_d.at[b]).wait()
        pltpu.async_copy(ones_v, sdeg_sh.at[sidx.at[b]], dsa_s.at[b],
                         add=True)
        pltpu.async_copy(ones_v, ddeg_sh.at[didx.at[b]], dsa_d.at[b],
                         add=True)
        if prefetch:
            nb = (b + 2) % 4
            if guard:
                pltpu.make_async_copy(ones_v, sdeg_sh.at[sidx.at[nb]],
                                      dsa_s.at[nb]).wait()
                pltpu.make_async_copy(ones_v, ddeg_sh.at[didx.at[nb]],
                                      dsa_d.at[nb]).wait()

            def _pf(fts, egs, agg_o, ind_o):
                dg_src_d(egs, ch + 2, nb, dsi_s.at[nb]).start()
                dg_dst_d(egs, ch + 2, nb, dsi_d.at[nb]).start()

            percore(_pf)

    for ch in range(DEG_CHUNKS - 2):
        deg_chunk(ch, ch % 4, prefetch=True, guard=(ch >= 2))
    deg_chunk(DEG_CHUNKS - 2, (DEG_CHUNKS - 2) % 4, prefetch=False, guard=False)
    deg_chunk(DEG_CHUNKS - 1, (DEG_CHUNKS - 1) % 4, prefetch=False, guard=False)
    for b in range(4):
        pltpu.make_async_copy(ones_v, sdeg_sh.at[sidx.at[b]],
                              dsa_s.at[b]).wait()
        pltpu.make_async_copy(ones_v, ddeg_sh.at[didx.at[b]],
                              dsa_d.at[b]).wait()

    def e_src_d(egs, ch, b, sem):
        return pltpu.make_async_copy(
            egs.at[0, pl.ds(base + ch * EK, EK)], es_v.at[b], sem)

    def e_dst_d(egs, ch, q, sem):
        return pltpu.make_async_copy(
            egs.at[1, pl.ds(base + ch * EK, EK)], ed_v.at[q], sem)

    def _edge_prologue(fts, egs, agg_o, ind_o):
        for b in range(2):
            e_src_d(egs, b, b, si_s.at[b]).start()
            e_dst_d(egs, b, b, si_d.at[b]).start()

    percore(_edge_prologue)
    plsc.subcore_barrier()

    def _indeg_wb(fts, egs, agg_o, ind_o):
        @pl.when(s < WB_TILES)
        def _():
            o = s * WB_ROWS
            pltpu.sync_copy(ddeg_sh.at[pl.ds(o, WB_ROWS)],
                            ind_o.at[0, pl.ds(o, WB_ROWS)])

    percore(_indeg_wb)

    def scale_rows(fts, row0, nrows, nrm0):
        for k in range(nrows // SK):
            r0 = row0 + k * SK
            pltpu.sync_copy(fts.at[pl.ds(r0, SK)], rows_v.at[0, pl.ds(0, SK)])

            @pl.loop(0, SK)
            def _(r):
                g = plsc.load_gather(
                    nrm_v, [jnp.full((16,), nrm0 + k * SK + r, jnp.int32)])
                for f in range(F_IN // 16):
                    sl = pl.ds(f * 16, 16)
                    rows_v[0, r, sl] = rows_v[0, r, sl] * g

            pltpu.sync_copy(rows_v.at[0, pl.ds(0, SK)], h_sh.at[pl.ds(r0, SK)])

    r0 = s * RT
    pltpu.sync_copy(sdeg_sh.at[pl.ds(r0, RT)], nrm_v)

    @pl.loop(0, RT, step=16)
    def _(i):
        nrm_v[pl.ds(i, 16)] = _newton_rsqrt16(nrm_v[pl.ds(i, 16)])

    def _scale_main(fts, egs, agg_o, ind_o):
        scale_rows(fts, r0, RT, 0)

    percore(_scale_main)

    @pl.when(s < 2)
    def _():
        re0 = NS * RT + s * SK
        pltpu.sync_copy(sdeg_sh.at[pl.ds(re0, SK)], nrm_v.at[pl.ds(0, SK)])

        @pl.loop(0, SK, step=16)
        def _(i):
            nrm_v[pl.ds(i, 16)] = _newton_rsqrt16(nrm_v[pl.ds(i, 16)])

        def _scale_extra(fts, egs, agg_o, ind_o):
            scale_rows(fts, re0, SK, 0)

        percore(_scale_extra)

    plsc.subcore_barrier()

    def edge_chunk(ch, b, guard, prefetch):
        br = b % 2
        q = b
        e_src_d(edges0, ch, br, si_s.at[br]).wait()
        e_dst_d(edges0, ch, q, si_d.at[q]).wait()

        if guard:
            @pl.when(ch >= 2)
            def _():
                pltpu.make_async_copy(
                    rows_v.at[br], agg_sh.at[ed_v.at[(q + 2) % 4]],
                    ss.at[br]).wait()
        else:
            pltpu.make_async_copy(
                rows_v.at[br], agg_sh.at[ed_v.at[(q + 2) % 4]],
                ss.at[br]).wait()

        pltpu.async_copy(h_sh.at[es_v.at[br]], rows_v.at[br],
                         sg.at[br]).wait()
        pltpu.async_copy(rows_v.at[br], agg_sh.at[ed_v.at[q]], ss.at[br],
                         add=True)

        if prefetch:
            def _pf(fts, egs, agg_o, ind_o):
                e_src_d(egs, ch + 2, br, si_s.at[br]).start()
                e_dst_d(egs, ch + 2, (q + 2) % 4,
                        si_d.at[(q + 2) % 4]).start()

            percore(_pf)

    _EMAIN = ((ECHUNKS - 2) // 4) * 4

    @pl.loop(0, _EMAIN, step=4)
    def _(i):
        for b in range(4):
            edge_chunk(i + b, b, guard=True, prefetch=True)

    for ch in range(_EMAIN, ECHUNKS):
        edge_chunk(ch, ch % 4, guard=False, prefetch=(ch + 2 < ECHUNKS))

    for b in range(2):
        pltpu.make_async_copy(
            rows_v.at[b], agg_sh.at[ed_v.at[b]], ss.at[b]).wait()
    plsc.subcore_barrier()

    def _agg_wb(fts, egs, agg_o, ind_o):
        @pl.when(s < WB_TILES)
        def _():
            r0 = s * WB_ROWS
            pltpu.sync_copy(agg_sh.at[pl.ds(r0, WB_ROWS)],
                            agg_o.at[pl.ds(r0, WB_ROWS)])

    percore(_agg_wb)


_fused_call = pl.kernel(
    _fused_body,
    out_type=(jax.ShapeDtypeStruct((N, F_IN), jnp.float32),
              jax.ShapeDtypeStruct((N, F_IN), jnp.float32),
              jax.ShapeDtypeStruct((1, N), jnp.float32),
              jax.ShapeDtypeStruct((1, N), jnp.float32)),
    mesh=_mesh,
    scratch_types=[
        pltpu.VMEM((2, EK), jnp.int32),
        pltpu.VMEM((4, EK), jnp.int32),
        pltpu.VMEM((2, EK, F_IN), jnp.float32),
        pltpu.VMEM((4, DEG_K), jnp.int32),
        pltpu.VMEM((4, DEG_K), jnp.int32),
        pltpu.VMEM((DEG_K,), jnp.float32),
        pltpu.VMEM((RT,), jnp.float32),
        pltpu.VMEM_SHARED((N, F_IN), jnp.float32),
        pltpu.VMEM_SHARED((N, F_IN), jnp.float32),
        pltpu.VMEM_SHARED((N_PAD,), jnp.float32),
        pltpu.VMEM_SHARED((N_PAD,), jnp.float32),
        pltpu.SemaphoreType.DMA((2,)),
        pltpu.SemaphoreType.DMA((4,)),
        pltpu.SemaphoreType.DMA((2,)),
        pltpu.SemaphoreType.DMA((2,)),
        pltpu.SemaphoreType.DMA((4,)),
        pltpu.SemaphoreType.DMA((4,)),
        pltpu.SemaphoreType.DMA((4,)),
        pltpu.SemaphoreType.DMA((4,)),
    ],
    compiler_params=_sc_params,
)


RB = 2048
NB = (N + RB - 1) // RB


def _out_body(agg_ref, indeg_ref, w_ref, b_ref, out_ref):
    nd = lax.rsqrt(jnp.maximum(indeg_ref[0], 1.0))
    a = agg_ref[...] * nd[:, None]
    out_ref[...] = lax.dot_general(
        a, w_ref[...], (((1,), (0,)), ((), ())),
        preferred_element_type=jnp.float32,
        precision=lax.Precision.HIGHEST) + b_ref[0][None, :]


_out_call = pl.pallas_call(
    _out_body,
    grid=(NB,),
    in_specs=[
        pl.BlockSpec((RB, F_IN), lambda r: (r, 0)),
        pl.BlockSpec((1, RB), lambda r: (0, r)),
        pl.BlockSpec((F_IN, F_OUT), lambda r: (0, 0)),
        pl.BlockSpec((1, F_OUT), lambda r: (0, 0)),
    ],
    out_specs=pl.BlockSpec((RB, F_OUT), lambda r: (r, 0)),
    out_shape=jax.ShapeDtypeStruct((N, F_OUT), jnp.float32),
)


def kernel(feats0, feats1, W, b, edge_index0, edge_index1):
    zeros1d = jnp.zeros((N_PAD,), jnp.float32)
    zeros2d = jnp.zeros((N, F_IN), jnp.float32)
    agg0, agg1, indeg0, indeg1 = _fused_call(
        feats0, feats1, edge_index0, edge_index1, zeros1d, zeros2d)
    b2 = b.reshape(1, F_OUT)
    out0 = _out_call(agg0, indeg0, W, b2)
    out1 = _out_call(agg1, indeg1, W, b2)
    return out0, out1


# speedup vs baseline: 1.0153x; 1.0153x over previous
import jax
import jax.numpy as jnp
from jax import lax
from jax.experimental import pallas as pl
from jax.experimental.pallas import tpu as pltpu
from jax.experimental.pallas import tpu_sc as plsc

N = 10000
E = 320000
F_IN = 64
F_OUT = 128

NC = 2
NS = 16
N_PAD = 10240

EPT = E // NS
DEG_K = 2000
DEG_CHUNKS = EPT // DEG_K
EK = 200
ECHUNKS = EPT // EK
SK = 200

RT = 600
WB_TILES = 10
WB_ROWS = N // WB_TILES

_mesh = plsc.VectorSubcoreMesh(
    core_axis_name="c", subcore_axis_name="s", num_cores=NC, num_subcores=NS)

_sc_params = pltpu.CompilerParams(use_tc_tiling_on_sc=False,
                                  needs_layout_passes=False)


def _newton_rsqrt16(v):
    x = jnp.maximum(v, 1.0)
    i = plsc.bitcast(x, jnp.int32)
    i = jnp.int32(0x5F3759DF) - lax.shift_right_logical(i, 1)
    y = plsc.bitcast(i, jnp.float32)
    for _ in range(3):
        y = y * (1.5 - 0.5 * x * y * y)
    return y


def _fused_body(feats0, feats1, edges0, edges1, zeros1d, zeros2d,
                agg0_out, agg1_out, indeg0_out, indeg1_out,
                es_v, ed_v, rows_v, sidx, didx, ones_v, nrm_v,
                h_sh, agg_sh, sdeg_sh, ddeg_sh,
                si_s, si_d, sg, ss, dsi_s, dsi_d, dsa_s, dsa_d):
    c = lax.axis_index("c")
    s = lax.axis_index("s")
    base = s * EPT

    def percore(fn):
        @pl.when(c == 0)
        def _():
            fn(feats0, edges0, agg0_out, indeg0_out)

        @pl.when(c == 1)
        def _():
            fn(feats1, edges1, agg1_out, indeg1_out)

    def dg_src_d(egs, ch, b, sem):
        return pltpu.make_async_copy(
            egs.at[0, pl.ds(base + ch * DEG_K, DEG_K)], sidx.at[b], sem)

    def dg_dst_d(egs, ch, b, sem):
        return pltpu.make_async_copy(
            egs.at[1, pl.ds(base + ch * DEG_K, DEG_K)], didx.at[b], sem)

    def _deg_prologue(fts, egs, agg_o, ind_o):
        for b in range(2):
            dg_src_d(egs, b, b, dsi_s.at[b]).start()
            dg_dst_d(egs, b, b, dsi_d.at[b]).start()

    percore(_deg_prologue)

    zn = N_PAD // NS
    z0 = s * zn
    pltpu.sync_copy(zeros1d.at[pl.ds(z0, zn)], sdeg_sh.at[pl.ds(z0, zn)])
    pltpu.sync_copy(zeros1d.at[pl.ds(z0, zn)], ddeg_sh.at[pl.ds(z0, zn)])

    @pl.when(s < WB_TILES)
    def _():
        r0 = s * WB_ROWS
        pltpu.sync_copy(zeros2d.at[pl.ds(r0, WB_ROWS)],
                        agg_sh.at[pl.ds(r0, WB_ROWS)])

    @pl.loop(0, DEG_K, step=16)
    def _(i):
        ones_v[pl.ds(i, 16)] = jnp.full((16,), 1.0, jnp.float32)

    plsc.subcore_barrier()

    def deg_chunk(ch, b, prefetch, guard):
        dg_src_d(edges0, ch, b, dsi_s.at[b]).wait()
        dg_dst_d(edges0, ch, b, dsi_d.at[b]).wait()
        pltpu.async_copy(ones_v, sdeg_sh.at[sidx.at[b]], dsa_s.at[b],
                         add=True)
        pltpu.async_copy(ones_v, ddeg_sh.at[didx.at[b]], dsa_d.at[b],
                         add=True)
        if prefetch:
            nb = (b + 2) % 4
            if guard:
                pltpu.make_async_copy(ones_v, sdeg_sh.at[sidx.at[nb]],
                                      dsa_s.at[nb]).wait()
                pltpu.make_async_copy(ones_v, ddeg_sh.at[didx.at[nb]],
                                      dsa_d.at[nb]).wait()

            def _pf(fts, egs, agg_o, ind_o):
                dg_src_d(egs, ch + 2, nb, dsi_s.at[nb]).start()
                dg_dst_d(egs, ch + 2, nb, dsi_d.at[nb]).start()

            percore(_pf)

    for ch in range(DEG_CHUNKS - 2):
        deg_chunk(ch, ch % 4, prefetch=True, guard=(ch >= 2))
    deg_chunk(DEG_CHUNKS - 2, (DEG_CHUNKS - 2) % 4, prefetch=False, guard=False)
    deg_chunk(DEG_CHUNKS - 1, (DEG_CHUNKS - 1) % 4, prefetch=False, guard=False)
    for b in range(4):
        pltpu.make_async_copy(ones_v, sdeg_sh.at[sidx.at[b]],
                              dsa_s.at[b]).wait()
        pltpu.make_async_copy(ones_v, ddeg_sh.at[didx.at[b]],
                              dsa_d.at[b]).wait()

    def e_src_d(egs, ch, b, sem):
        return pltpu.make_async_copy(
            egs.at[0, pl.ds(base + ch * EK, EK)], es_v.at[b], sem)

    def e_dst_d(egs, ch, q, sem):
        return pltpu.make_async_copy(
            egs.at[1, pl.ds(base + ch * EK, EK)], ed_v.at[q], sem)

    def _edge_prologue(fts, egs, agg_o, ind_o):
        for b in range(2):
            e_src_d(egs, b, b, si_s.at[b]).start()
            e_dst_d(egs, b, b, si_d.at[b]).start()

    percore(_edge_prologue)
    plsc.subcore_barrier()

    def _indeg_wb(fts, egs, agg_o, ind_o):
        @pl.when(s < WB_TILES)
        def _():
            o = s * WB_ROWS
            pltpu.sync_copy(ddeg_sh.at[pl.ds(o, WB_ROWS)],
                            ind_o.at[0, pl.ds(o, WB_ROWS)])

    percore(_indeg_wb)

    def scale_rows(fts, row0, nrows, nrm0):
        for k in range(nrows // SK):
            r0 = row0 + k * SK
            pltpu.sync_copy(fts.at[pl.ds(r0, SK)], rows_v.at[0, pl.ds(0, SK)])

            @pl.loop(0, SK)
            def _(r):
                g = plsc.load_gather(
                    nrm_v, [jnp.full((16,), nrm0 + k * SK + r, jnp.int32)])
                for f in range(F_IN // 16):
                    sl = pl.ds(f * 16, 16)
                    rows_v[0, r, sl] = rows_v[0, r, sl] * g

            pltpu.sync_copy(rows_v.at[0, pl.ds(0, SK)], h_sh.at[pl.ds(r0, SK)])

    r0 = s * RT
    pltpu.sync_copy(sdeg_sh.at[pl.ds(r0, RT)], nrm_v)

    NKC = RT // SK

    def x_in_d(fts_, k, b):
        return pltpu.make_async_copy(
            fts_.at[pl.ds(r0 + k * SK, SK)], rows_v.at[b], sg.at[b])

    def h_out_d(k, b):
        return pltpu.make_async_copy(
            rows_v.at[b], h_sh.at[pl.ds(r0 + k * SK, SK)], ss.at[b])

    def _sc0(fts, egs, agg_o, ind_o):
        x_in_d(fts, 0, 0).start()

    percore(_sc0)

    @pl.loop(0, RT, step=16)
    def _(i):
        nrm_v[pl.ds(i, 16)] = _newton_rsqrt16(nrm_v[pl.ds(i, 16)])

    def scale_compute(b, koff):
        @pl.loop(0, SK)
        def _(r):
            g = plsc.load_gather(nrm_v, [jnp.full((16,), koff + r, jnp.int32)])
            for f in range(F_IN // 16):
                sl = pl.ds(f * 16, 16)
                rows_v[b, r, sl] = rows_v[b, r, sl] * g

    for k in range(NKC):
        b = k % 2
        x_in_d(feats0, k, b).wait()
        if k + 1 < NKC:
            if k >= 1:
                h_out_d(k - 1, 1 - b).wait()

            def _sk(fts, egs, agg_o, ind_o, _k=k, _b=b):
                x_in_d(fts, _k + 1, 1 - _b).start()

            percore(_sk)
        scale_compute(b, k * SK)
        h_out_d(k, b).start()
    h_out_d(NKC - 2, (NKC - 2) % 2).wait()
    h_out_d(NKC - 1, (NKC - 1) % 2).wait()

    @pl.when(s < 2)
    def _():
        re0 = NS * RT + s * SK
        pltpu.sync_copy(sdeg_sh.at[pl.ds(re0, SK)], nrm_v.at[pl.ds(0, SK)])

        @pl.loop(0, SK, step=16)
        def _(i):
            nrm_v[pl.ds(i, 16)] = _newton_rsqrt16(nrm_v[pl.ds(i, 16)])

        def _scale_extra(fts, egs, agg_o, ind_o):
            scale_rows(fts, re0, SK, 0)

        percore(_scale_extra)

    plsc.subcore_barrier()

    def edge_chunk(ch, b, guard, prefetch):
        br = b % 2
        q = b
        e_src_d(edges0, ch, br, si_s.at[br]).wait()
        e_dst_d(edges0, ch, q, si_d.at[q]).wait()

        if guard:
            @pl.when(ch >= 2)
            def _():
                pltpu.make_async_copy(
                    rows_v.at[br], agg_sh.at[ed_v.at[(q + 2) % 4]],
                    ss.at[br]).wait()
        else:
            pltpu.make_async_copy(
                rows_v.at[br], agg_sh.at[ed_v.at[(q + 2) % 4]],
                ss.at[br]).wait()

        pltpu.async_copy(h_sh.at[es_v.at[br]], rows_v.at[br],
                         sg.at[br]).wait()
        pltpu.async_copy(rows_v.at[br], agg_sh.at[ed_v.at[q]], ss.at[br],
                         add=True)

        if prefetch:
            def _pf(fts, egs, agg_o, ind_o):
                e_src_d(egs, ch + 2, br, si_s.at[br]).start()
                e_dst_d(egs, ch + 2, (q + 2) % 4,
                        si_d.at[(q + 2) % 4]).start()

            percore(_pf)

    _EMAIN = ((ECHUNKS - 2) // 4) * 4

    @pl.loop(0, _EMAIN, step=4)
    def _(i):
        for b in range(4):
            edge_chunk(i + b, b, guard=True, prefetch=True)

    for ch in range(_EMAIN, ECHUNKS):
        edge_chunk(ch, ch % 4, guard=False, prefetch=(ch + 2 < ECHUNKS))

    for b in range(2):
        pltpu.make_async_copy(
            rows_v.at[b], agg_sh.at[ed_v.at[b]], ss.at[b]).wait()
    plsc.subcore_barrier()

    def _agg_wb(fts, egs, agg_o, ind_o):
        @pl.when(s < WB_TILES)
        def _():
            r0 = s * WB_ROWS
            pltpu.sync_copy(agg_sh.at[pl.ds(r0, WB_ROWS)],
                            agg_o.at[pl.ds(r0, WB_ROWS)])

    percore(_agg_wb)


_fused_call = pl.kernel(
    _fused_body,
    out_type=(jax.ShapeDtypeStruct((N, F_IN), jnp.float32),
              jax.ShapeDtypeStruct((N, F_IN), jnp.float32),
              jax.ShapeDtypeStruct((1, N), jnp.float32),
              jax.ShapeDtypeStruct((1, N), jnp.float32)),
    mesh=_mesh,
    scratch_types=[
        pltpu.VMEM((2, EK), jnp.int32),
        pltpu.VMEM((4, EK), jnp.int32),
        pltpu.VMEM((2, EK, F_IN), jnp.float32),
        pltpu.VMEM((4, DEG_K), jnp.int32),
        pltpu.VMEM((4, DEG_K), jnp.int32),
        pltpu.VMEM((DEG_K,), jnp.float32),
        pltpu.VMEM((RT,), jnp.float32),
        pltpu.VMEM_SHARED((N, F_IN), jnp.float32),
        pltpu.VMEM_SHARED((N, F_IN), jnp.float32),
        pltpu.VMEM_SHARED((N_PAD,), jnp.float32),
        pltpu.VMEM_SHARED((N_PAD,), jnp.float32),
        pltpu.SemaphoreType.DMA((2,)),
        pltpu.SemaphoreType.DMA((4,)),
        pltpu.SemaphoreType.DMA((2,)),
        pltpu.SemaphoreType.DMA((2,)),
        pltpu.SemaphoreType.DMA((4,)),
        pltpu.SemaphoreType.DMA((4,)),
        pltpu.SemaphoreType.DMA((4,)),
        pltpu.SemaphoreType.DMA((4,)),
    ],
    compiler_params=_sc_params,
)


RB = 2048
NB = (N + RB - 1) // RB


def _out_body(agg_ref, indeg_ref, w_ref, b_ref, out_ref):
    nd = lax.rsqrt(jnp.maximum(indeg_ref[0], 1.0))
    a = agg_ref[...] * nd[:, None]
    out_ref[...] = lax.dot_general(
        a, w_ref[...], (((1,), (0,)), ((), ())),
        preferred_element_type=jnp.float32,
        precision=lax.Precision.HIGHEST) + b_ref[0][None, :]


_out_call = pl.pallas_call(
    _out_body,
    grid=(NB,),
    in_specs=[
        pl.BlockSpec((RB, F_IN), lambda r: (r, 0)),
        pl.BlockSpec((1, RB), lambda r: (0, r)),
        pl.BlockSpec((F_IN, F_OUT), lambda r: (0, 0)),
        pl.BlockSpec((1, F_OUT), lambda r: (0, 0)),
    ],
    out_specs=pl.BlockSpec((RB, F_OUT), lambda r: (r, 0)),
    out_shape=jax.ShapeDtypeStruct((N, F_OUT), jnp.float32),
)


def kernel(feats0, feats1, W, b, edge_index0, edge_index1):
    zeros1d = jnp.zeros((N_PAD,), jnp.float32)
    zeros2d = jnp.zeros((N, F_IN), jnp.float32)
    agg0, agg1, indeg0, indeg1 = _fused_call(
        feats0, feats1, edge_index0, edge_index1, zeros1d, zeros2d)
    b2 = b.reshape(1, F_OUT)
    out0 = _out_call(agg0, indeg0, W, b2)
    out1 = _out_call(agg1, indeg1, W, b2)
    return out0, out1
